# baseline (speedup 1.0000x reference)
import jax
import jax.numpy as jnp
from jax.experimental import pallas as pl
from jax.experimental.pallas import tpu as pltpu

_NUM_EXPERTS = 16
_TOP_K = 2
_HIDDEN = 2048
_FFN = 1408
_BLOCK = 128
_TOKENS = 32
_NF = _FFN // _BLOCK
_NK = _HIDDEN // _BLOCK
_KH = _HIDDEN // 2
_NKH = _NK // 2
_DH = _HIDDEN // 2


def _scale_row(sv, nblk):
    return jax.lax.broadcast_in_dim(sv, (nblk, _BLOCK), (0,)).reshape(
        1, nblk * _BLOCK)


def _proj_half(x_ref, gu_ref, sgu_ref, out_sref, khalf, fofs, accumulate):
    x = x_ref[:, khalf * _KH:(khalf + 1) * _KH]
    for f in range(_NF):
        sl = slice(f * _BLOCK, (f + 1) * _BLOCK)
        sg = _scale_row(
            sgu_ref[0, fofs + f, khalf * _NKH:(khalf + 1) * _NKH], _NKH)
        part = jax.lax.dot_general(
            x * sg, gu_ref[0, sl, :], (((1,), (1,)), ((), ())),
            preferred_element_type=jnp.float32)
        if accumulate:
            out_sref[:, sl] += part
        else:
            out_sref[:, sl] = part


def _fused(x_ref, gu_ref, sgu_ref, dn_ref, sdn_ref, ids_ref, wts_ref,
           o_ref, hg_ref, act_ref):
    e = pl.program_id(0)
    p = pl.program_id(1)

    @pl.when(p == 0)
    def _():
        _proj_half(x_ref, gu_ref, sgu_ref, hg_ref, 0, 0, False)

    @pl.when(p == 1)
    def _():
        _proj_half(x_ref, gu_ref, sgu_ref, hg_ref, 1, 0, True)

    @pl.when(p == 2)
    def _():
        _proj_half(x_ref, gu_ref, sgu_ref, act_ref, 0, _NF, False)

    @pl.when(p == 3)
    def _():
        x = x_ref[:, _KH:]
        for f in range(_NF):
            sl = slice(f * _BLOCK, (f + 1) * _BLOCK)
            su = _scale_row(sgu_ref[0, _NF + f, _NKH:], _NKH)
            hu = act_ref[:, sl] + jax.lax.dot_general(
                x * su, gu_ref[0, sl, :], (((1,), (1,)), ((), ())),
                preferred_element_type=jnp.float32)
            g = hg_ref[:, sl]
            act_ref[:, sl] = g * jax.nn.sigmoid(g) * hu

    @pl.when(p >= 4)
    def _down():
        dhalf = p - 4
        ids = ids_ref[...]
        wts = wts_ref[...]
        c = jnp.sum(jnp.where(ids == e, wts, 0.0), axis=1, keepdims=True)
        a = act_ref[...] * c
        for d in range(_NKH):
            sl = slice(d * _BLOCK, (d + 1) * _BLOCK)
            osl = pl.ds(dhalf * _DH + d * _BLOCK, _BLOCK)
            sr = _scale_row(sdn_ref[0, dhalf * _NKH + d, :], _NF)
            part = jax.lax.dot_general(
                a * sr, dn_ref[0, sl, :], (((1,), (1,)), ((), ())),
                preferred_element_type=jnp.float32)

            @pl.when(e == 0)
            def _():
                o_ref[:, osl] = part

            @pl.when(e != 0)
            def _():
                o_ref[:, osl] += part


def _gu_map(e, p):
    q = jnp.minimum(p, 3)
    return (e, q // 2, q % 2)


@jax.jit
def kernel(hidden_states, topk_weights, topk_ids, gate_up_weight,
           gate_up_scale, down_weight, down_scale):
    return pl.pallas_call(
        _fused,
        grid=(_NUM_EXPERTS, 6),
        in_specs=[
            pl.BlockSpec((_TOKENS, _HIDDEN), lambda e, p: (0, 0)),
            pl.BlockSpec((1, _FFN, _KH), _gu_map),
            pl.BlockSpec((1, 2 * _NF, _NK), lambda e, p: (e, 0, 0)),
            pl.BlockSpec((1, _DH, _FFN),
                         lambda e, p: (e, jnp.maximum(p - 4, 0), 0)),
            pl.BlockSpec((1, _NK, _NF), lambda e, p: (e, 0, 0)),
            pl.BlockSpec((_TOKENS, _TOP_K), lambda e, p: (0, 0)),
            pl.BlockSpec((_TOKENS, _TOP_K), lambda e, p: (0, 0)),
        ],
        out_specs=pl.BlockSpec((_TOKENS, _HIDDEN), lambda e, p: (0, 0)),
        out_shape=jax.ShapeDtypeStruct((_TOKENS, _HIDDEN), jnp.float32),
        scratch_shapes=[
            pltpu.VMEM((_TOKENS, _FFN), jnp.float32),
            pltpu.VMEM((_TOKENS, _FFN), jnp.float32),
        ],
        compiler_params=pltpu.CompilerParams(
            dimension_semantics=("arbitrary", "arbitrary")),
    )(hidden_states, gate_up_weight, gate_up_scale, down_weight,
      down_scale, topk_ids, topk_weights)


# speedup vs baseline: 1.0645x; 1.0645x over previous
import jax
import jax.numpy as jnp
from jax.experimental import pallas as pl
from jax.experimental.pallas import tpu as pltpu

_NUM_EXPERTS = 16
_TOP_K = 2
_HIDDEN = 2048
_FFN = 1408
_BLOCK = 128
_TOKENS = 32
_NF = _FFN // _BLOCK
_NK = _HIDDEN // _BLOCK


def _scale_row(sv, nblk):
    return jax.lax.broadcast_in_dim(sv, (nblk, _BLOCK), (0,)).reshape(
        1, nblk * _BLOCK)


def _stage1(x_ref, w_ref, s_ref, o_ref):
    x = x_ref[...]
    for f in range(_NF):
        sl = slice(f * _BLOCK, (f + 1) * _BLOCK)
        slu = slice(_FFN + f * _BLOCK, _FFN + (f + 1) * _BLOCK)
        sg = _scale_row(s_ref[0, f, :], _NK)
        su = _scale_row(s_ref[0, f + _NF, :], _NK)
        hg = jax.lax.dot_general(x * sg, w_ref[0, sl, :],
                                 (((1,), (1,)), ((), ())),
                                 preferred_element_type=jnp.float32)
        hu = jax.lax.dot_general(x * su, w_ref[0, slu, :],
                                 (((1,), (1,)), ((), ())),
                                 preferred_element_type=jnp.float32)
        o_ref[0, :, sl] = hg * jax.nn.sigmoid(hg) * hu


def _stage2(a_ref, wlo_ref, whi_ref, s_ref, ids_ref, wts_ref, o_ref):
    e = pl.program_id(0)
    ids = ids_ref[...]
    wts = wts_ref[...]
    c = jnp.sum(jnp.where(ids == e, wts, 0.0), axis=1, keepdims=True)
    a = a_ref[0] * c
    half = _NK // 2
    for d in range(_NK):
        sl = slice(d * _BLOCK, (d + 1) * _BLOCK)
        w_ref = wlo_ref if d < half else whi_ref
        wsl = slice((d % half) * _BLOCK, (d % half + 1) * _BLOCK)
        sr = _scale_row(s_ref[0, d, :], _NF)
        p = jax.lax.dot_general(a * sr, w_ref[0, wsl, :],
                                (((1,), (1,)), ((), ())),
                                preferred_element_type=jnp.float32)

        @pl.when(e == 0)
        def _():
            o_ref[:, sl] = p

        @pl.when(e != 0)
        def _():
            o_ref[:, sl] += p


@jax.jit
def kernel(hidden_states, topk_weights, topk_ids, gate_up_weight,
           gate_up_scale, down_weight, down_scale):
    act = pl.pallas_call(
        _stage1,
        grid=(_NUM_EXPERTS,),
        in_specs=[
            pl.BlockSpec((_TOKENS, _HIDDEN), lambda e: (0, 0)),
            pl.BlockSpec((1, 2 * _FFN, _HIDDEN), lambda e: (e, 0, 0)),
            pl.BlockSpec((1, 2 * _NF, _NK), lambda e: (e, 0, 0)),
        ],
        out_specs=pl.BlockSpec((1, _TOKENS, _FFN), lambda e: (e, 0, 0)),
        out_shape=jax.ShapeDtypeStruct((_NUM_EXPERTS, _TOKENS, _FFN),
                                       jnp.float32),
        compiler_params=pltpu.CompilerParams(
            dimension_semantics=("parallel",)),
    )(hidden_states, gate_up_weight, gate_up_scale)

    out = pl.pallas_call(
        _stage2,
        grid=(_NUM_EXPERTS,),
        in_specs=[
            pl.BlockSpec((1, _TOKENS, _FFN), lambda e: (e, 0, 0)),
            pl.BlockSpec((1, _HIDDEN // 2, _FFN), lambda e: (e, 0, 0)),
            pl.BlockSpec((1, _HIDDEN // 2, _FFN), lambda e: (e, 1, 0)),
            pl.BlockSpec((1, _NK, _NF), lambda e: (e, 0, 0)),
            pl.BlockSpec((_TOKENS, _TOP_K), lambda e: (0, 0)),
            pl.BlockSpec((_TOKENS, _TOP_K), lambda e: (0, 0)),
        ],
        out_specs=pl.BlockSpec((_TOKENS, _HIDDEN), lambda e: (0, 0)),
        out_shape=jax.ShapeDtypeStruct((_TOKENS, _HIDDEN), jnp.float32),
        compiler_params=pltpu.CompilerParams(
            dimension_semantics=("arbitrary",)),
    )(act, down_weight, down_weight, down_scale, topk_ids, topk_weights)
    return out
